# Initial kernel scaffold; baseline (speedup 1.0000x reference)
import jax
import jax.numpy as jnp
from jax.experimental import pallas as pl


def kernel(x, mask, Wf, Uf, bf, Wb, Ub, bb, Wout, bout):
    raise NotImplementedError("write your pallas kernel here")


# speedup vs baseline: 3.4139x; 3.4139x over previous
import jax
import jax.numpy as jnp
from jax.experimental import pallas as pl
from jax.experimental.pallas import tpu as pltpu


def _bigru_kernel(xs_ref, ms_ref, wf_ref, ufT_ref, bf_ref,
                  wb_ref, ubT_ref, bb_ref, wof_ref, wob_ref, bout_ref,
                  out_ref, h_ref, pf_ref):
    S = xs_ref.shape[0]
    H = ufT_ref.shape[1]

    wf = wf_ref[:, :]
    ufT = ufT_ref[:, :]
    bf = bf_ref[:, :]
    wof = wof_ref[:, :]

    h_ref[:, :] = jnp.zeros_like(h_ref)

    def fwd(t, carry):
        x_t = xs_ref[pl.ds(t, 1), :]
        h = h_ref[:, :]
        g = wf * x_t + bf
        gh = jnp.dot(ufT, h, preferred_element_type=jnp.float32)
        z = jax.nn.sigmoid(g[0:H, :] + gh[0:H, :])
        r = jax.nn.sigmoid(g[H:2 * H, :] + gh[H:2 * H, :])
        c = jnp.tanh(g[2 * H:3 * H, :] + r * gh[2 * H:3 * H, :])
        hn = (1.0 - z) * h + z * c
        h_ref[:, :] = hn
        pf_ref[pl.ds(t, 1), :] = jnp.sum(hn * wof, axis=0, keepdims=True)
        return carry

    jax.lax.fori_loop(0, S, fwd, 0)

    wb = wb_ref[:, :]
    ubT = ubT_ref[:, :]
    bb = bb_ref[:, :]
    wob = wob_ref[:, :]
    bout = bout_ref[0, 0]

    h_ref[:, :] = jnp.zeros_like(h_ref)

    def bwd(i, carry):
        t = S - 1 - i
        x_t = xs_ref[pl.ds(t, 1), :]
        h = h_ref[:, :]
        g = wb * x_t + bb
        gh = jnp.dot(ubT, h, preferred_element_type=jnp.float32)
        z = jax.nn.sigmoid(g[0:H, :] + gh[0:H, :])
        r = jax.nn.sigmoid(g[H:2 * H, :] + gh[H:2 * H, :])
        c = jnp.tanh(g[2 * H:3 * H, :] + r * gh[2 * H:3 * H, :])
        hn = (1.0 - z) * h + z * c
        h_ref[:, :] = hn
        pb = jnp.sum(hn * wob, axis=0, keepdims=True)
        imp = pf_ref[pl.ds(t, 1), :] + pb + bout
        m = ms_ref[pl.ds(t, 1), :]
        out_ref[pl.ds(t, 1), :] = m * x_t + (1.0 - m) * imp
        return carry

    jax.lax.fori_loop(0, S, bwd, 0)


def kernel(x, mask, Wf, Uf, bf, Wb, Ub, bb, Wout, bout):
    B, S, N, C = x.shape
    H = Uf.shape[0]
    R = B * N
    RT = 2048
    G = R // RT

    xs = x.transpose(1, 0, 2, 3).reshape(S, R)
    ms = mask.astype(jnp.float32).transpose(1, 0, 2, 3).reshape(S, R)

    wf = Wf.reshape(3 * H, 1)
    wb = Wb.reshape(3 * H, 1)
    ufT = Uf.T
    ubT = Ub.T
    bf2 = bf.reshape(3 * H, 1)
    bb2 = bb.reshape(3 * H, 1)
    wof = Wout[:H, 0:1]
    wob = Wout[H:, 0:1]
    bout2 = bout.reshape(1, 1)

    full = lambda shape: pl.BlockSpec(shape, lambda i: (0, 0))
    tile = pl.BlockSpec((S, RT), lambda i: (0, i))

    out = pl.pallas_call(
        _bigru_kernel,
        grid=(G,),
        in_specs=[
            tile,
            tile,
            full((3 * H, 1)),
            full((3 * H, H)),
            full((3 * H, 1)),
            full((3 * H, 1)),
            full((3 * H, H)),
            full((3 * H, 1)),
            full((H, 1)),
            full((H, 1)),
            full((1, 1)),
        ],
        out_specs=tile,
        out_shape=jax.ShapeDtypeStruct((S, R), jnp.float32),
        scratch_shapes=[
            pltpu.VMEM((H, RT), jnp.float32),
            pltpu.VMEM((S, RT), jnp.float32),
        ],
        compiler_params=pltpu.CompilerParams(
            dimension_semantics=("arbitrary",),
        ),
    )(xs, ms, wf, ufT, bf2, wb, ubT, bb2, wof, wob, bout2)

    return out.reshape(S, B, N, C).transpose(1, 0, 2, 3)
